# baseline (device time: 12789 ns/iter reference)
import functools
import os

import jax
import jax.numpy as jnp
from jax import lax
from jax.experimental import pallas as pl
from jax.experimental.pallas import tpu as pltpu

N_DEV = 4
B = 64
D = 512

N_EXCH = int(os.environ.get("KERNEL_EXCH", "6"))

_PREC = {
    "default": jax.lax.Precision.DEFAULT,
    "high": jax.lax.Precision.HIGH,
    "highest": jax.lax.Precision.HIGHEST,
}[os.environ.get("KERNEL_PREC", "highest")]


def kernel(x, Win0, Wout0, Win1, Wout1, Win2, Wout2):
    def body(
        x_ref,
        win0_ref,
        wout0_ref,
        win1_ref,
        wout1_ref,
        win2_ref,
        wout2_ref,
        out_ref,
        send_buf,
        comm_ref,
        send_sems,
        recv_sems,
    ):
        my = lax.axis_index("i")
        p1 = my ^ 1
        p2 = 3 - my

        if N_EXCH > 0:
            barrier_sem = pltpu.get_barrier_semaphore()
            for p in (p1, p2):
                pl.semaphore_signal(
                    barrier_sem, inc=1, device_id=(p,),
                    device_id_type=pl.DeviceIdType.MESH,
                )

        wins = [win0_ref, win1_ref, win2_ref]
        wouts = [wout0_ref, wout1_ref, wout2_ref]

        pending_sends = []
        xcur = x_ref[:, :]
        for layer in range(3):
            h = jnp.maximum(
                jnp.dot(xcur, wins[layer][:, :], precision=_PREC,
                        preferred_element_type=jnp.float32),
                0.0,
            )
            part = jnp.dot(h, wouts[layer][:, :], precision=_PREC,
                           preferred_element_type=jnp.float32)
            if layer == 0 and N_EXCH > 0:
                pl.semaphore_wait(barrier_sem, 2)
            for stage, partner in enumerate((p1, p2)):
                e = 2 * layer + stage
                if e >= N_EXCH:
                    continue
                send_buf[e, :, :] = part
                rdma = pltpu.make_async_remote_copy(
                    src_ref=send_buf.at[e],
                    dst_ref=comm_ref.at[e],
                    send_sem=send_sems.at[e],
                    recv_sem=recv_sems.at[e],
                    device_id=(partner,),
                    device_id_type=pl.DeviceIdType.MESH,
                )
                rdma.start()
                pending_sends.append(rdma)
                rdma.wait_recv()
                part = part + comm_ref[e, :, :]
            xcur = part

        comm_ref[0, :, :] = xcur
        out_ref[:, :] = comm_ref[0, pl.ds(my * (B // N_DEV), B // N_DEV), :]

        for rdma in pending_sends:
            rdma.wait_send()

        if N_EXCH > 0:
            @functools.partial(
                pl.run_scoped, second_barrier=pltpu.SemaphoreType.REGULAR
            )
            def _(second_barrier):
                for p in (p1, p2):
                    pl.semaphore_signal(
                        second_barrier, inc=1, device_id=(p,),
                        device_id_type=pl.DeviceIdType.MESH,
                    )
                pl.semaphore_wait(second_barrier, 2)

    return pl.pallas_call(
        body,
        out_shape=jax.ShapeDtypeStruct((B // N_DEV, D), jnp.float32),
        in_specs=[pl.BlockSpec(memory_space=pltpu.VMEM)] * 7,
        out_specs=pl.BlockSpec(memory_space=pltpu.VMEM),
        scratch_shapes=[
            pltpu.VMEM((6, B, D), jnp.float32),
            pltpu.VMEM((6, B, D), jnp.float32),
            pltpu.SemaphoreType.DMA((6,)),
            pltpu.SemaphoreType.DMA((6,)),
        ],
        compiler_params=(
            pltpu.CompilerParams(collective_id=0)
            if N_EXCH > 0
            else pltpu.CompilerParams()
        ),
    )(x, Win0, Wout0, Win1, Wout1, Win2, Wout2)


# device time: 11111 ns/iter; 1.1510x vs baseline; 1.1510x over previous
import functools
import os

import jax
import jax.numpy as jnp
from jax import lax
from jax.experimental import pallas as pl
from jax.experimental.pallas import tpu as pltpu

N_DEV = 4
B = 64
D = 512

N_EXCH = int(os.environ.get("KERNEL_EXCH", "6"))

_PREC = {
    "default": jax.lax.Precision.DEFAULT,
    "high": jax.lax.Precision.HIGH,
    "highest": jax.lax.Precision.HIGHEST,
}[os.environ.get("KERNEL_PREC", "highest")]

MODE = os.environ.get("KERNEL_MODE", "full")


def kernel(x, Win0, Wout0, Win1, Wout1, Win2, Wout2):
    if MODE == "empty":
        def empty_body(x_ref, *refs):
            out_ref = refs[6]
            my = lax.axis_index("i")
            out_ref[:, :] = x_ref[pl.ds(my * (B // N_DEV), B // N_DEV), :]

        return pl.pallas_call(
            empty_body,
            out_shape=jax.ShapeDtypeStruct((B // N_DEV, D), jnp.float32),
            in_specs=[pl.BlockSpec(memory_space=pltpu.VMEM)]
            + [pl.BlockSpec(memory_space=pltpu.MemorySpace.HBM)] * 6,
            out_specs=pl.BlockSpec(memory_space=pltpu.VMEM),
        )(x, Win0, Wout0, Win1, Wout1, Win2, Wout2)

    def body(
        x_ref,
        win0_ref,
        wout0_ref,
        win1_ref,
        wout1_ref,
        win2_ref,
        wout2_ref,
        out_ref,
        send_buf,
        comm_ref,
        send_sems,
        recv_sems,
    ):
        my = lax.axis_index("i")
        p1 = my ^ 1
        p2 = 3 - my

        if N_EXCH > 0:
            barrier_sem = pltpu.get_barrier_semaphore()
            for p in (p1, p2):
                pl.semaphore_signal(
                    barrier_sem, inc=1, device_id=(p,),
                    device_id_type=pl.DeviceIdType.MESH,
                )

        wins = [win0_ref, win1_ref, win2_ref]
        wouts = [wout0_ref, wout1_ref, wout2_ref]

        pending_sends = []
        xcur = x_ref[:, :]
        for layer in range(3):
            h = jnp.maximum(
                jnp.dot(xcur, wins[layer][:, :], precision=_PREC,
                        preferred_element_type=jnp.float32),
                0.0,
            )
            part = jnp.dot(h, wouts[layer][:, :], precision=_PREC,
                           preferred_element_type=jnp.float32)
            if layer == 0 and N_EXCH > 0:
                pl.semaphore_wait(barrier_sem, 2)
            for stage, partner in enumerate((p1, p2)):
                e = 2 * layer + stage
                if e >= N_EXCH:
                    continue
                send_buf[e, :, :] = part
                rdma = pltpu.make_async_remote_copy(
                    src_ref=send_buf.at[e],
                    dst_ref=comm_ref.at[e],
                    send_sem=send_sems.at[e],
                    recv_sem=recv_sems.at[e],
                    device_id=(partner,),
                    device_id_type=pl.DeviceIdType.MESH,
                )
                rdma.start()
                pending_sends.append(rdma)
                rdma.wait_recv()
                part = part + comm_ref[e, :, :]
            xcur = part

        comm_ref[0, :, :] = xcur
        out_ref[:, :] = comm_ref[0, pl.ds(my * (B // N_DEV), B // N_DEV), :]

        for rdma in pending_sends:
            rdma.wait_send()

        if N_EXCH > 0:
            @functools.partial(
                pl.run_scoped, second_barrier=pltpu.SemaphoreType.REGULAR
            )
            def _(second_barrier):
                for p in (p1, p2):
                    pl.semaphore_signal(
                        second_barrier, inc=1, device_id=(p,),
                        device_id_type=pl.DeviceIdType.MESH,
                    )
                pl.semaphore_wait(second_barrier, 2)

    return pl.pallas_call(
        body,
        out_shape=jax.ShapeDtypeStruct((B // N_DEV, D), jnp.float32),
        in_specs=[pl.BlockSpec(memory_space=pltpu.VMEM)] * 7,
        out_specs=pl.BlockSpec(memory_space=pltpu.VMEM),
        scratch_shapes=[
            pltpu.VMEM((6, B, D), jnp.float32),
            pltpu.VMEM((6, B, D), jnp.float32),
            pltpu.SemaphoreType.DMA((6,)),
            pltpu.SemaphoreType.DMA((6,)),
        ],
        compiler_params=(
            pltpu.CompilerParams(collective_id=0)
            if N_EXCH > 0
            else pltpu.CompilerParams()
        ),
    )(x, Win0, Wout0, Win1, Wout1, Win2, Wout2)
